# baseline (speedup 1.0000x reference)
import jax
import jax.numpy as jnp
from jax.experimental import pallas as pl

_N_CODES = 1024
_DIM = 32
_COMMIT = 0.25


def _vq_body(z_ref, w_ref, w8_ref, out_ref, loss_ref):
    b = pl.program_id(0)
    zb = z_ref[0]
    w = w_ref[...]
    zbt = zb.T
    wsq = jnp.sum(w * w, axis=1)
    zsq = jnp.sum(zbt * zbt, axis=1)
    mm = jax.lax.dot_general(zbt, w, (((1,), (1,)), ((), ())),
                             preferred_element_type=jnp.float32)
    dists = (zsq[:, None] + wsq[None, :]) - 2.0 * mm
    dmin = jnp.min(dists, axis=1, keepdims=True)
    lane = jax.lax.broadcasted_iota(jnp.int32, dists.shape, 1)
    idx = jnp.min(jnp.where(dists == dmin, lane, _N_CODES), axis=1)
    hw = idx.shape[0]
    hi = idx // 8
    lo = idx - 8 * hi
    ohg = (jax.lax.broadcasted_iota(jnp.int32, (hw, _N_CODES // 8), 1)
           == hi[:, None]).astype(jnp.float32)
    part = jax.lax.dot_general(ohg, w8_ref[...], (((1,), (0,)), ((), ())),
                               preferred_element_type=jnp.float32)
    zq = part[:, 0:_DIM]
    for j in range(1, 8):
        zq = jnp.where((lo == j)[:, None], part[:, j * _DIM:(j + 1) * _DIM], zq)
    zqt = zq.T
    diff = zqt - zb
    out_ref[0] = zb + diff

    @pl.when(b == 0)
    def _():
        loss_ref[...] = jnp.zeros_like(loss_ref)

    loss_ref[...] += jnp.sum(diff * diff, keepdims=True)


def kernel(z, W):
    B, C, H, Wd = z.shape
    HW = H * Wd
    z3 = z.reshape(B, C, HW)
    out, loss = pl.pallas_call(
        _vq_body,
        grid=(B,),
        in_specs=[
            pl.BlockSpec((1, C, HW), lambda b: (b, 0, 0)),
            pl.BlockSpec((_N_CODES, _DIM), lambda b: (0, 0)),
            pl.BlockSpec((_N_CODES // 8, _DIM * 8), lambda b: (0, 0)),
        ],
        out_specs=[
            pl.BlockSpec((1, C, HW), lambda b: (b, 0, 0)),
            pl.BlockSpec((1, 1), lambda b: (0, 0)),
        ],
        out_shape=[
            jax.ShapeDtypeStruct((B, C, HW), jnp.float32),
            jax.ShapeDtypeStruct((1, 1), jnp.float32),
        ],
    )(z3, W, W.reshape(_N_CODES // 8, _DIM * 8))
    m = loss[0, 0] / (B * C * H * Wd)
    vq_loss = m + _COMMIT * m
    return out.reshape(B, C, H, Wd), vq_loss


# speedup vs baseline: 1.0769x; 1.0769x over previous
import jax
import jax.numpy as jnp
from jax.experimental import pallas as pl

_N_CODES = 1024
_DIM = 32
_COMMIT = 0.25


def _vq_body(z_ref, w_ref, out_ref, loss_ref):
    b = pl.program_id(0)
    zb = z_ref[0]
    w = w_ref[...]
    zbt = zb.T
    wsq = jnp.sum(w * w, axis=1)
    zsq = jnp.sum(zbt * zbt, axis=1)
    mm2 = jax.lax.dot_general(zbt + zbt, w, (((1,), (1,)), ((), ())),
                              preferred_element_type=jnp.float32)
    dists = (zsq[:, None] + wsq[None, :]) - mm2
    dmin = jnp.min(dists, axis=1, keepdims=True)
    lane = jax.lax.broadcasted_iota(jnp.int32, dists.shape, 1)
    idx = jnp.min(jnp.where(dists == dmin, lane, _N_CODES), axis=1)
    oh = (lane == idx[:, None]).astype(jnp.float32)
    zq = jax.lax.dot_general(oh, w, (((1,), (0,)), ((), ())),
                             preferred_element_type=jnp.float32)
    zqt = zq.T
    diff = zqt - zb
    out_ref[0] = zb + diff

    @pl.when(b == 0)
    def _():
        loss_ref[...] = jnp.zeros_like(loss_ref)

    loss_ref[...] += jnp.sum(diff * diff, keepdims=True)


def kernel(z, W):
    B, C, H, Wd = z.shape
    HW = H * Wd
    z3 = z.reshape(B, C, HW)
    out, loss = pl.pallas_call(
        _vq_body,
        grid=(B,),
        in_specs=[
            pl.BlockSpec((1, C, HW), lambda b: (b, 0, 0)),
            pl.BlockSpec((_N_CODES, _DIM), lambda b: (0, 0)),
        ],
        out_specs=[
            pl.BlockSpec((1, C, HW), lambda b: (b, 0, 0)),
            pl.BlockSpec((1, 1), lambda b: (0, 0)),
        ],
        out_shape=[
            jax.ShapeDtypeStruct((B, C, HW), jnp.float32),
            jax.ShapeDtypeStruct((1, 1), jnp.float32),
        ],
    )(z3, W)
    m = loss[0, 0] / (B * C * H * Wd)
    vq_loss = m + _COMMIT * m
    return out.reshape(B, C, H, Wd), vq_loss


# speedup vs baseline: 1.2484x; 1.1592x over previous
import jax
import jax.numpy as jnp
from jax.experimental import pallas as pl

_N_CODES = 1024
_DIM = 32
_COMMIT = 0.25


_CHUNKS = 2


def _vq_body(z_ref, w_ref, out_ref, loss_ref):
    b = pl.program_id(0)
    zb = z_ref[0]
    w = w_ref[...]
    wsq = jnp.sum(w * w, axis=1)
    hw = zb.shape[1]
    cw = hw // _CHUNKS
    psum = None
    for s in range(_CHUNKS):
        zc = zb[:, s * cw:(s + 1) * cw]
        zct = zc.T
        zsq = jnp.sum(zct * zct, axis=1)
        mm2 = jax.lax.dot_general(zct + zct, w, (((1,), (1,)), ((), ())),
                                  preferred_element_type=jnp.float32)
        dists = (zsq[:, None] + wsq[None, :]) - mm2
        dmin = jnp.min(dists, axis=1, keepdims=True)
        lane = jax.lax.broadcasted_iota(jnp.int32, dists.shape, 1)
        idx = jnp.min(jnp.where(dists == dmin, lane, _N_CODES), axis=1)
        oh = (lane == idx[:, None]).astype(jnp.float32)
        zq = jax.lax.dot_general(oh, w, (((1,), (0,)), ((), ())),
                                 preferred_element_type=jnp.float32)
        zqt = zq.T
        diff = zqt - zc
        out_ref[0, :, s * cw:(s + 1) * cw] = zc + diff
        p = jnp.sum(diff * diff, keepdims=True)
        psum = p if psum is None else psum + p

    @pl.when(b == 0)
    def _():
        loss_ref[...] = jnp.zeros_like(loss_ref)

    loss_ref[...] += psum


def kernel(z, W):
    B, C, H, Wd = z.shape
    HW = H * Wd
    z3 = z.reshape(B, C, HW)
    out, loss = pl.pallas_call(
        _vq_body,
        grid=(B,),
        in_specs=[
            pl.BlockSpec((1, C, HW), lambda b: (b, 0, 0)),
            pl.BlockSpec((_N_CODES, _DIM), lambda b: (0, 0)),
        ],
        out_specs=[
            pl.BlockSpec((1, C, HW), lambda b: (b, 0, 0)),
            pl.BlockSpec((1, 1), lambda b: (0, 0)),
        ],
        out_shape=[
            jax.ShapeDtypeStruct((B, C, HW), jnp.float32),
            jax.ShapeDtypeStruct((1, 1), jnp.float32),
        ],
    )(z3, W)
    m = loss[0, 0] / (B * C * H * Wd)
    vq_loss = m + _COMMIT * m
    return out.reshape(B, C, H, Wd), vq_loss


# speedup vs baseline: 1.2543x; 1.0047x over previous
import jax
import jax.numpy as jnp
from jax.experimental import pallas as pl

_N_CODES = 1024
_DIM = 32
_COMMIT = 0.25


_CHUNKS = 4


def _vq_body(z_ref, w_ref, out_ref, loss_ref):
    b = pl.program_id(0)
    zb = z_ref[0]
    w = w_ref[...]
    wsq = jnp.sum(w * w, axis=1)
    hw = zb.shape[1]
    cw = hw // _CHUNKS
    psum = None
    for s in range(_CHUNKS):
        zc = zb[:, s * cw:(s + 1) * cw]
        zct = zc.T
        zsq = jnp.sum(zct * zct, axis=1)
        mm2 = jax.lax.dot_general(zct + zct, w, (((1,), (1,)), ((), ())),
                                  preferred_element_type=jnp.float32)
        dists = (zsq[:, None] + wsq[None, :]) - mm2
        dmin = jnp.min(dists, axis=1, keepdims=True)
        lane = jax.lax.broadcasted_iota(jnp.int32, dists.shape, 1)
        idx = jnp.min(jnp.where(dists == dmin, lane, _N_CODES), axis=1)
        oh = (lane == idx[:, None]).astype(jnp.float32)
        zq = jax.lax.dot_general(oh, w, (((1,), (0,)), ((), ())),
                                 preferred_element_type=jnp.float32)
        zqt = zq.T
        diff = zqt - zc
        out_ref[0, :, s * cw:(s + 1) * cw] = zc + diff
        p = jnp.sum(diff * diff, keepdims=True)
        psum = p if psum is None else psum + p

    @pl.when(b == 0)
    def _():
        loss_ref[...] = jnp.zeros_like(loss_ref)

    loss_ref[...] += psum


def kernel(z, W):
    B, C, H, Wd = z.shape
    HW = H * Wd
    z3 = z.reshape(B, C, HW)
    out, loss = pl.pallas_call(
        _vq_body,
        grid=(B,),
        in_specs=[
            pl.BlockSpec((1, C, HW), lambda b: (b, 0, 0)),
            pl.BlockSpec((_N_CODES, _DIM), lambda b: (0, 0)),
        ],
        out_specs=[
            pl.BlockSpec((1, C, HW), lambda b: (b, 0, 0)),
            pl.BlockSpec((1, 1), lambda b: (0, 0)),
        ],
        out_shape=[
            jax.ShapeDtypeStruct((B, C, HW), jnp.float32),
            jax.ShapeDtypeStruct((1, 1), jnp.float32),
        ],
    )(z3, W)
    m = loss[0, 0] / (B * C * H * Wd)
    vq_loss = m + _COMMIT * m
    return out.reshape(B, C, H, Wd), vq_loss


# speedup vs baseline: 1.4400x; 1.1481x over previous
import jax
import jax.numpy as jnp
from jax.experimental import pallas as pl

_N_CODES = 1024
_DIM = 32
_COMMIT = 0.25


_CHUNKS = 4
_BATCHES_PER_STEP = 2


def _vq_body(z_ref, w_ref, out_ref, loss_ref):
    b = pl.program_id(0)
    w = w_ref[...]
    wsq = jnp.sum(w * w, axis=1)
    nb = z_ref.shape[0]
    hw = z_ref.shape[2]
    cw = (nb * hw) // _CHUNKS
    psum = None
    for s in range(_CHUNKS):
        bi, so = divmod(s * cw, hw)
        zc = z_ref[bi, :, so:so + cw]
        zct = zc.T
        zsq = jnp.sum(zct * zct, axis=1)
        mm2 = jax.lax.dot_general(zct + zct, w, (((1,), (1,)), ((), ())),
                                  preferred_element_type=jnp.float32)
        dists = (zsq[:, None] + wsq[None, :]) - mm2
        dmin = jnp.min(dists, axis=1, keepdims=True)
        lane = jax.lax.broadcasted_iota(jnp.int32, dists.shape, 1)
        idx = jnp.min(jnp.where(dists == dmin, lane, _N_CODES), axis=1)
        oh = (lane == idx[:, None]).astype(jnp.float32)
        zq = jax.lax.dot_general(oh, w, (((1,), (0,)), ((), ())),
                                 preferred_element_type=jnp.float32)
        zqt = zq.T
        diff = zqt - zc
        out_ref[bi, :, so:so + cw] = zc + diff
        p = jnp.sum(diff * diff, keepdims=True)
        psum = p if psum is None else psum + p

    @pl.when(b == 0)
    def _():
        loss_ref[...] = jnp.zeros_like(loss_ref)

    loss_ref[...] += psum


def kernel(z, W):
    B, C, H, Wd = z.shape
    HW = H * Wd
    z3 = z.reshape(B, C, HW)
    nb = _BATCHES_PER_STEP
    out, loss = pl.pallas_call(
        _vq_body,
        grid=(B // nb,),
        in_specs=[
            pl.BlockSpec((nb, C, HW), lambda b: (b, 0, 0)),
            pl.BlockSpec((_N_CODES, _DIM), lambda b: (0, 0)),
        ],
        out_specs=[
            pl.BlockSpec((nb, C, HW), lambda b: (b, 0, 0)),
            pl.BlockSpec((1, 1), lambda b: (0, 0)),
        ],
        out_shape=[
            jax.ShapeDtypeStruct((B, C, HW), jnp.float32),
            jax.ShapeDtypeStruct((1, 1), jnp.float32),
        ],
    )(z3, W)
    m = loss[0, 0] / (B * C * H * Wd)
    vq_loss = m + _COMMIT * m
    return out.reshape(B, C, H, Wd), vq_loss


# speedup vs baseline: 1.5619x; 1.0847x over previous
import jax
import jax.numpy as jnp
from jax.experimental import pallas as pl

_N_CODES = 1024
_DIM = 32
_COMMIT = 0.25


_CHUNKS = 8
_BATCHES_PER_STEP = 4


def _vq_body(z_ref, w_ref, out_ref, loss_ref):
    b = pl.program_id(0)
    w = w_ref[...]
    wsq = jnp.sum(w * w, axis=1)
    nb = z_ref.shape[0]
    hw = z_ref.shape[2]
    cw = (nb * hw) // _CHUNKS
    psum = None
    for s in range(_CHUNKS):
        bi, so = divmod(s * cw, hw)
        zc = z_ref[bi, :, so:so + cw]
        zct = zc.T
        zsq = jnp.sum(zct * zct, axis=1)
        mm2 = jax.lax.dot_general(zct + zct, w, (((1,), (1,)), ((), ())),
                                  preferred_element_type=jnp.float32)
        dists = (zsq[:, None] + wsq[None, :]) - mm2
        dmin = jnp.min(dists, axis=1, keepdims=True)
        lane = jax.lax.broadcasted_iota(jnp.int32, dists.shape, 1)
        idx = jnp.min(jnp.where(dists == dmin, lane, _N_CODES), axis=1)
        oh = (lane == idx[:, None]).astype(jnp.float32)
        zq = jax.lax.dot_general(oh, w, (((1,), (0,)), ((), ())),
                                 preferred_element_type=jnp.float32)
        zqt = zq.T
        diff = zqt - zc
        out_ref[bi, :, so:so + cw] = zc + diff
        p = jnp.sum(diff * diff, keepdims=True)
        psum = p if psum is None else psum + p

    @pl.when(b == 0)
    def _():
        loss_ref[...] = jnp.zeros_like(loss_ref)

    loss_ref[...] += psum


def kernel(z, W):
    B, C, H, Wd = z.shape
    HW = H * Wd
    z3 = z.reshape(B, C, HW)
    nb = _BATCHES_PER_STEP
    out, loss = pl.pallas_call(
        _vq_body,
        grid=(B // nb,),
        in_specs=[
            pl.BlockSpec((nb, C, HW), lambda b: (b, 0, 0)),
            pl.BlockSpec((_N_CODES, _DIM), lambda b: (0, 0)),
        ],
        out_specs=[
            pl.BlockSpec((nb, C, HW), lambda b: (b, 0, 0)),
            pl.BlockSpec((1, 1), lambda b: (0, 0)),
        ],
        out_shape=[
            jax.ShapeDtypeStruct((B, C, HW), jnp.float32),
            jax.ShapeDtypeStruct((1, 1), jnp.float32),
        ],
    )(z3, W)
    m = loss[0, 0] / (B * C * H * Wd)
    vq_loss = m + _COMMIT * m
    return out.reshape(B, C, H, Wd), vq_loss


# speedup vs baseline: 1.6005x; 1.0247x over previous
import jax
import jax.numpy as jnp
from jax.experimental import pallas as pl

_N_CODES = 1024
_DIM = 32
_COMMIT = 0.25


_CHUNKS = 16
_BATCHES_PER_STEP = 8


def _vq_body(z_ref, w_ref, out_ref, loss_ref):
    b = pl.program_id(0)
    w = w_ref[...]
    wsq = jnp.sum(w * w, axis=1)
    nb = z_ref.shape[0]
    hw = z_ref.shape[2]
    cw = (nb * hw) // _CHUNKS
    psum = None
    for s in range(_CHUNKS):
        bi, so = divmod(s * cw, hw)
        zc = z_ref[bi, :, so:so + cw]
        zct = zc.T
        zsq = jnp.sum(zct * zct, axis=1)
        mm2 = jax.lax.dot_general(zct + zct, w, (((1,), (1,)), ((), ())),
                                  preferred_element_type=jnp.float32)
        dists = (zsq[:, None] + wsq[None, :]) - mm2
        dmin = jnp.min(dists, axis=1, keepdims=True)
        lane = jax.lax.broadcasted_iota(jnp.int32, dists.shape, 1)
        idx = jnp.min(jnp.where(dists == dmin, lane, _N_CODES), axis=1)
        oh = (lane == idx[:, None]).astype(jnp.float32)
        zq = jax.lax.dot_general(oh, w, (((1,), (0,)), ((), ())),
                                 preferred_element_type=jnp.float32)
        zqt = zq.T
        diff = zqt - zc
        out_ref[bi, :, so:so + cw] = zc + diff
        p = jnp.sum(diff * diff, keepdims=True)
        psum = p if psum is None else psum + p

    @pl.when(b == 0)
    def _():
        loss_ref[...] = jnp.zeros_like(loss_ref)

    loss_ref[...] += psum


def kernel(z, W):
    B, C, H, Wd = z.shape
    HW = H * Wd
    z3 = z.reshape(B, C, HW)
    nb = _BATCHES_PER_STEP
    out, loss = pl.pallas_call(
        _vq_body,
        grid=(B // nb,),
        in_specs=[
            pl.BlockSpec((nb, C, HW), lambda b: (b, 0, 0)),
            pl.BlockSpec((_N_CODES, _DIM), lambda b: (0, 0)),
        ],
        out_specs=[
            pl.BlockSpec((nb, C, HW), lambda b: (b, 0, 0)),
            pl.BlockSpec((1, 1), lambda b: (0, 0)),
        ],
        out_shape=[
            jax.ShapeDtypeStruct((B, C, HW), jnp.float32),
            jax.ShapeDtypeStruct((1, 1), jnp.float32),
        ],
    )(z3, W)
    m = loss[0, 0] / (B * C * H * Wd)
    vq_loss = m + _COMMIT * m
    return out.reshape(B, C, H, Wd), vq_loss


# speedup vs baseline: 1.6045x; 1.0025x over previous
import jax
import jax.numpy as jnp
from jax.experimental import pallas as pl

_N_CODES = 1024
_DIM = 32
_COMMIT = 0.25


_CHUNKS = 16
_BATCHES_PER_STEP = 8


def _vq_body(z_ref, w_ref, out_ref, loss_ref):
    b = pl.program_id(0)
    w = w_ref[...]
    wsq = jnp.sum(w * w, axis=1)
    nb = z_ref.shape[0]
    hw = z_ref.shape[2]
    cw = (nb * hw) // _CHUNKS
    lane = jax.lax.broadcasted_iota(jnp.int32, (cw, _N_CODES), 1)
    psum = None
    for s in range(_CHUNKS):
        bi, so = divmod(s * cw, hw)
        zc = z_ref[bi, :, so:so + cw]
        zct = zc.T
        zsq = jnp.sum(zct * zct, axis=1)
        mm2 = jax.lax.dot_general(zct + zct, w, (((1,), (1,)), ((), ())),
                                  preferred_element_type=jnp.float32)
        dists = (zsq[:, None] + wsq[None, :]) - mm2
        dmin = jnp.min(dists, axis=1, keepdims=True)
        idx = jnp.min(jnp.where(dists == dmin, lane, _N_CODES), axis=1)
        oh = (lane == idx[:, None]).astype(jnp.float32)
        zq = jax.lax.dot_general(oh, w, (((1,), (0,)), ((), ())),
                                 preferred_element_type=jnp.float32)
        zqt = zq.T
        diff = zqt - zc
        out_ref[bi, :, so:so + cw] = zc + diff
        p = jnp.sum(diff * diff, keepdims=True)
        psum = p if psum is None else psum + p

    @pl.when(b == 0)
    def _():
        loss_ref[...] = jnp.zeros_like(loss_ref)

    loss_ref[...] += psum


def kernel(z, W):
    B, C, H, Wd = z.shape
    HW = H * Wd
    z3 = z.reshape(B, C, HW)
    nb = _BATCHES_PER_STEP
    out, loss = pl.pallas_call(
        _vq_body,
        grid=(B // nb,),
        in_specs=[
            pl.BlockSpec((nb, C, HW), lambda b: (b, 0, 0)),
            pl.BlockSpec((_N_CODES, _DIM), lambda b: (0, 0)),
        ],
        out_specs=[
            pl.BlockSpec((nb, C, HW), lambda b: (b, 0, 0)),
            pl.BlockSpec((1, 1), lambda b: (0, 0)),
        ],
        out_shape=[
            jax.ShapeDtypeStruct((B, C, HW), jnp.float32),
            jax.ShapeDtypeStruct((1, 1), jnp.float32),
        ],
    )(z3, W)
    m = loss[0, 0] / (B * C * H * Wd)
    vq_loss = m + _COMMIT * m
    return out.reshape(B, C, H, Wd), vq_loss
